# baseline (speedup 1.0000x reference)
import jax
import jax.numpy as jnp
import numpy as np
from jax import lax
from jax.experimental import pallas as pl
from jax.experimental.pallas import tpu as pltpu
from jax.experimental.pallas import tpu_sc as plsc

VOCAB = 1000
D_MODEL = 1024
MAX_SEQ = 2048
BATCH = 4

_KB = 3
_TB = BATCH - _KB

_NTOK = BATCH * MAX_SEQ
_INFO = plsc.get_sparse_core_info()
_NC, _NS, _L = _INFO.num_cores, _INFO.num_subcores, _INFO.num_lanes
_NW = _NC * _NS
_PPW = MAX_SEQ // _NW
_C = 16
_NCHUNK = _KB * _PPW // _C
_NBUF = 5
_DEPTH = 3
_NGRP = _PPW // _C
_VP = VOCAB
_RB = 256


def _pos_encoding():
    even_i = np.arange(0, D_MODEL, 2, dtype=np.float32)
    denominator = np.power(np.float32(10000.0), even_i / np.float32(D_MODEL))
    position = np.arange(MAX_SEQ, dtype=np.float32).reshape(MAX_SEQ, 1)
    even_pe = np.sin(position / denominator, dtype=np.float32)
    odd_pe = np.cos(position / denominator, dtype=np.float32)
    return np.stack([even_pe, odd_pe], axis=2).reshape(MAX_SEQ, D_MODEL)


_PE_NP = _pos_encoding()


_mesh = plsc.VectorSubcoreMesh(core_axis_name="c", subcore_axis_name="s")


@pl.kernel(
    mesh=_mesh,
    out_type=jax.ShapeDtypeStruct((_NTOK, D_MODEL), jnp.float32),
    scratch_types=[
        pltpu.VMEM((_KB * _PPW,), jnp.int32),
        pltpu.VMEM((_NBUF, _C, D_MODEL), jnp.float32),
        pltpu.VMEM((2, _C * D_MODEL), jnp.float32),
        pltpu.SemaphoreType.DMA,
        pltpu.SemaphoreType.DMA,
        pltpu.SemaphoreType.DMA,
    ],
)
def _sc_kernel(x_hbm, table_hbm, pe_hbm, out_hbm,
               idx_v, rbuf, pbuf, sem_i, sem_g, sem_w):
    sid = lax.axis_index("s")
    wid = sid * _NC + lax.axis_index("c")
    pos0 = wid * _PPW

    idx_cp = [
        pltpu.async_copy(
            x_hbm.at[pl.ds(b * MAX_SEQ + pos0, _PPW)],
            idx_v.at[pl.ds(b * _PPW, _PPW)],
            sem_i,
        )
        for b in range(_KB)
    ]
    for cp in idx_cp:
        cp.wait()

    def idx_slice(ci):
        h, b = ci // _KB, ci % _KB
        return idx_v.at[pl.ds(b * _PPW + h * _C, _C)]

    def out_slice(ci):
        h, b = ci // _KB, ci % _KB
        return out_hbm.at[pl.ds(b * MAX_SEQ + pos0 + h * _C, _C)]

    def start_gather(ci):
        return pltpu.async_copy(
            table_hbm.at[idx_slice(ci)], rbuf.at[ci % _NBUF], sem_g
        )

    def start_pe(g):
        return pltpu.async_copy(
            pe_hbm.at[pl.ds((pos0 + g * _C) * D_MODEL, _C * D_MODEL)],
            pbuf.at[g % 2],
            sem_i,
        )

    gat = [None] * _NCHUNK
    wr = [None] * _NCHUNK
    pe_cp = [None] * _NGRP
    for ci in range(_DEPTH):
        gat[ci] = start_gather(ci)
    pe_cp[0] = start_pe(0)

    for ci in range(_NCHUNK):
        g = ci // _KB
        if ci + _DEPTH < _NCHUNK:
            if ci >= _NBUF - _DEPTH:
                wr[ci - (_NBUF - _DEPTH)].wait()
            gat[ci + _DEPTH] = start_gather(ci + _DEPTH)
        if ci % _KB == 0:
            pe_cp[g].wait()
            if g + 1 < _NGRP:
                pe_cp[g + 1] = start_pe(g + 1)
        gat[ci].wait()
        buf = rbuf.at[ci % _NBUF]
        pv = pbuf.at[g % 2]

        def row_body(r, carry):
            for j in range(D_MODEL // _L):
                sl = pl.ds(j * _L, _L)
                buf[r, sl] = buf[r, sl] + pv[pl.ds(r * D_MODEL + j * _L, _L)]
            return carry

        lax.fori_loop(0, _C, row_body, 0)
        wr[ci] = pltpu.async_copy(buf, out_slice(ci), sem_w)

    for ci in range(max(0, _NCHUNK - _NBUF), _NCHUNK):
        wr[ci].wait()


def _tc_body(idx_ref, tbl_ref, pe_ref, out_ref):
    ids = idx_ref[...]
    iot = lax.broadcasted_iota(jnp.int32, (_RB, _VP), 1)
    onehot = (iot == ids).astype(jnp.float32)
    rows = lax.dot_general(
        onehot, tbl_ref[...], (((1,), (0,)), ((), ())),
        precision=lax.Precision.HIGHEST,
        preferred_element_type=jnp.float32,
    )
    out_ref[...] = rows + pe_ref[...]


def _tc_kernel(x_tc, table_pad, pe):
    nblk = MAX_SEQ // _RB
    return pl.pallas_call(
        _tc_body,
        grid=(_TB * nblk,),
        in_specs=[
            pl.BlockSpec((_RB, 1), lambda i: (i, 0)),
            pl.BlockSpec((_VP, D_MODEL), lambda i: (0, 0)),
            pl.BlockSpec((_RB, D_MODEL), lambda i: (i % nblk, 0)),
        ],
        out_specs=pl.BlockSpec((_RB, D_MODEL), lambda i: (i, 0)),
        out_shape=jax.ShapeDtypeStruct((_TB * MAX_SEQ, D_MODEL), jnp.float32),
    )(x_tc, table_pad, pe)


@jax.jit
def _run(xf, emb_table):
    pe_flat = jnp.asarray(_PE_NP.reshape(-1))
    pe = jnp.asarray(_PE_NP)
    sc_out = _sc_kernel(xf, emb_table, pe_flat)
    x_tc = xf[_KB * MAX_SEQ:].reshape(_TB * MAX_SEQ, 1)
    tc_out = _tc_kernel(x_tc, emb_table, pe)
    return lax.dynamic_update_slice(sc_out, tc_out, (_KB * MAX_SEQ, 0))


def kernel(x, emb_table):
    out = _run(x.reshape(_NTOK).astype(jnp.int32), emb_table)
    return out.reshape(BATCH, MAX_SEQ, D_MODEL)


# speedup vs baseline: 1.4533x; 1.4533x over previous
import jax
import jax.numpy as jnp
import numpy as np
from jax import lax
from jax.experimental import pallas as pl
from jax.experimental.pallas import tpu as pltpu
from jax.experimental.pallas import tpu_sc as plsc

VOCAB = 1000
D_MODEL = 1024
MAX_SEQ = 2048
BATCH = 4

_KB = 3
_TB = BATCH - _KB

_NTOK = BATCH * MAX_SEQ
_INFO = plsc.get_sparse_core_info()
_NC, _NS, _L = _INFO.num_cores, _INFO.num_subcores, _INFO.num_lanes
_NW = _NC * _NS
_PPW = MAX_SEQ // _NW
_C = 16
_NCHUNK = _KB * _PPW // _C
_NBUF = 5
_DEPTH = 3
_NGRP = _PPW // _C
_VP = VOCAB
_RB = 256


def _pos_encoding():
    even_i = np.arange(0, D_MODEL, 2, dtype=np.float32)
    denominator = np.power(np.float32(10000.0), even_i / np.float32(D_MODEL))
    position = np.arange(MAX_SEQ, dtype=np.float32).reshape(MAX_SEQ, 1)
    even_pe = np.sin(position / denominator, dtype=np.float32)
    odd_pe = np.cos(position / denominator, dtype=np.float32)
    return np.stack([even_pe, odd_pe], axis=2).reshape(MAX_SEQ, D_MODEL)


_PE_NP = _pos_encoding()


_mesh = plsc.VectorSubcoreMesh(core_axis_name="c", subcore_axis_name="s")


@pl.kernel(
    mesh=_mesh,
    out_type=jax.ShapeDtypeStruct((_NTOK, D_MODEL), jnp.float32),
    scratch_types=[
        pltpu.VMEM((_KB * _PPW,), jnp.int32),
        pltpu.VMEM((_NBUF, _C, D_MODEL), jnp.float32),
        pltpu.VMEM((2, _C, D_MODEL), jnp.float32),
        pltpu.SemaphoreType.DMA,
        pltpu.SemaphoreType.DMA,
        pltpu.SemaphoreType.DMA,
    ],
)
def _sc_kernel(x_hbm, table_hbm, pe_hbm, out_hbm,
               idx_v, rbuf, pbuf, sem_i, sem_g, sem_w):
    sid = lax.axis_index("s")
    wid = sid * _NC + lax.axis_index("c")
    pos0 = wid * _PPW

    idx_cp = [
        pltpu.async_copy(
            x_hbm.at[pl.ds(b * MAX_SEQ + pos0, _PPW)],
            idx_v.at[pl.ds(b * _PPW, _PPW)],
            sem_i,
        )
        for b in range(_KB)
    ]
    for cp in idx_cp:
        cp.wait()

    def idx_slice(ci):
        h, b = ci // _KB, ci % _KB
        return idx_v.at[pl.ds(b * _PPW + h * _C, _C)]

    def out_slice(ci):
        h, b = ci // _KB, ci % _KB
        return out_hbm.at[pl.ds(b * MAX_SEQ + pos0 + h * _C, _C)]

    def start_gather(ci):
        return pltpu.async_copy(
            table_hbm.at[idx_slice(ci)], rbuf.at[ci % _NBUF], sem_g
        )

    def start_pe(g):
        return pltpu.async_copy(
            pe_hbm.at[pl.ds(pos0 + g * _C, _C)], pbuf.at[g % 2], sem_i
        )

    gat = [None] * _NCHUNK
    wr = [None] * _NCHUNK
    pe_cp = [None] * _NGRP
    for ci in range(_DEPTH):
        gat[ci] = start_gather(ci)
    pe_cp[0] = start_pe(0)

    for ci in range(_NCHUNK):
        g = ci // _KB
        if ci + _DEPTH < _NCHUNK:
            if ci >= _NBUF - _DEPTH:
                wr[ci - (_NBUF - _DEPTH)].wait()
            gat[ci + _DEPTH] = start_gather(ci + _DEPTH)
        if ci % _KB == 0:
            pe_cp[g].wait()
            if g + 1 < _NGRP:
                pe_cp[g + 1] = start_pe(g + 1)
        gat[ci].wait()
        buf = rbuf.at[ci % _NBUF]
        pv = pbuf.at[g % 2]

        def row_body(r, carry):
            for j in range(D_MODEL // _L):
                sl = pl.ds(j * _L, _L)
                buf[r, sl] = buf[r, sl] + pv[r, sl]
            return carry

        lax.fori_loop(0, _C, row_body, 0)
        wr[ci] = pltpu.async_copy(buf, out_slice(ci), sem_w)

    for ci in range(max(0, _NCHUNK - _NBUF), _NCHUNK):
        wr[ci].wait()


def _tc_body(idx_ref, tbl_ref, pe_ref, out_ref):
    ids = idx_ref[...]
    iot = lax.broadcasted_iota(jnp.int32, (_RB, _VP), 1)
    onehot = (iot == ids).astype(jnp.float32)
    rows = lax.dot_general(
        onehot, tbl_ref[...], (((1,), (0,)), ((), ())),
        precision=lax.Precision.HIGHEST,
        preferred_element_type=jnp.float32,
    )
    out_ref[...] = rows + pe_ref[...]


def _tc_kernel(x_tc, table_pad, pe):
    nblk = MAX_SEQ // _RB
    return pl.pallas_call(
        _tc_body,
        grid=(_TB * nblk,),
        in_specs=[
            pl.BlockSpec((_RB, 1), lambda i: (i, 0)),
            pl.BlockSpec((_VP, D_MODEL), lambda i: (0, 0)),
            pl.BlockSpec((_RB, D_MODEL), lambda i: (i % nblk, 0)),
        ],
        out_specs=pl.BlockSpec((_RB, D_MODEL), lambda i: (i, 0)),
        out_shape=jax.ShapeDtypeStruct((_TB * MAX_SEQ, D_MODEL), jnp.float32),
    )(x_tc, table_pad, pe)


@jax.jit
def _run(xf, emb_table):
    pe = jnp.asarray(_PE_NP)
    sc_out = _sc_kernel(xf, emb_table, pe)
    x_tc = xf[_KB * MAX_SEQ:].reshape(_TB * MAX_SEQ, 1)
    tc_out = _tc_kernel(x_tc, emb_table, pe)
    return lax.dynamic_update_slice(sc_out, tc_out, (_KB * MAX_SEQ, 0))


def kernel(x, emb_table):
    out = _run(x.reshape(_NTOK).astype(jnp.int32), emb_table)
    return out.reshape(BATCH, MAX_SEQ, D_MODEL)
